# Initial kernel scaffold; baseline (speedup 1.0000x reference)
import jax
import jax.numpy as jnp
from jax.experimental import pallas as pl


def kernel(pos, edge_index, batch, W_spline, root, conv_bias, bn_gamma, bn_beta, W1, b1, W2, b2):
    raise NotImplementedError("write your pallas kernel here")


# speedup vs baseline: 27.4941x; 27.4941x over previous
import jax
import jax.numpy as jnp
from jax.experimental import pallas as pl

B = 8
P = 1024
N = B * P
KNN = 20
L = 10
F = 64
KS = 5
NC = 40


def _knn_clusters_kernel(pos_ref, out_ref):
    x = pos_ref[0]
    row = jax.lax.broadcasted_iota(jnp.int32, (P, P), 0)
    col = jax.lax.broadcasted_iota(jnp.int32, (P, P), 1)
    d2 = jnp.zeros((P, P), dtype=jnp.float32)
    for d in range(3):
        xd = x[:, d:d + 1]
        diff = xd - xd.T
        d2 = d2 + diff * diff
    d2 = d2 + jnp.where(row == col, jnp.float32(1e10), jnp.float32(0.0))

    cols = col
    parts = []
    for _ in range(KNN):
        m = jnp.min(d2, axis=1, keepdims=True)
        cand = jnp.where(d2 == m, cols, jnp.int32(P))
        idx = jnp.min(cand, axis=1, keepdims=True)
        onehot = (cols == idx).astype(jnp.float32)
        nbr = jax.lax.dot_general(
            onehot, x, (((1,), (0,)), ((), ())),
            preferred_element_type=jnp.float32)
        parts.append(x - nbr)
        d2 = d2 + onehot * jnp.float32(1e10)
    out_ref[0] = jnp.concatenate(parts, axis=1)


def _frames_kernel(cl_ref, out_ref):
    c = [[cl_ref[k * 3 + d] for d in range(3)] for k in range(KNN)]

    def acc(a, b):
        s = c[0][a] * c[0][b]
        for k in range(1, L):
            s = s + c[k][a] * c[k][b]
        return s

    A = {}
    A[(0, 0)] = acc(0, 0); A[(1, 1)] = acc(1, 1); A[(2, 2)] = acc(2, 2)
    A[(0, 1)] = acc(0, 1); A[(0, 2)] = acc(0, 2); A[(1, 2)] = acc(1, 2)

    one = jnp.ones_like(A[(0, 0)])
    zero = jnp.zeros_like(one)
    V = [[one, zero, zero], [zero, one, zero], [zero, zero, one]]

    for _ in range(5):
        for (p, q) in ((0, 1), (0, 2), (1, 2)):
            r = 3 - p - q
            app = A[(p, p)]; aqq = A[(q, q)]; apq = A[(p, q)]
            apr = A[(min(p, r), max(p, r))]
            aqr = A[(min(q, r), max(q, r))]
            th = 0.5 * jnp.arctan2(2.0 * apq, aqq - app)
            cth = jnp.cos(th); sth = jnp.sin(th)
            cc = cth * cth; ss = sth * sth; cs = cth * sth
            new_pp = cc * app - 2.0 * cs * apq + ss * aqq
            new_qq = ss * app + 2.0 * cs * apq + cc * aqq
            new_pq = cs * (app - aqq) + (cc - ss) * apq
            new_pr = cth * apr - sth * aqr
            new_qr = sth * apr + cth * aqr
            A[(p, p)] = new_pp
            A[(q, q)] = new_qq
            A[(p, q)] = new_pq
            A[(min(p, r), max(p, r))] = new_pr
            A[(min(q, r), max(q, r))] = new_qr
            for i in range(3):
                vip = V[i][p]; viq = V[i][q]
                V[i][p] = cth * vip - sth * viq
                V[i][q] = sth * vip + cth * viq

    dc = [[c[k][0] * V[0][b] + c[k][1] * V[1][b] + c[k][2] * V[2][b]
           for b in range(3)] for k in range(KNN)]

    s2 = dc[0][2]
    for k in range(1, KNN):
        s2 = s2 + dc[k][2]
    sgn = jnp.sign(s2)
    for k in range(KNN):
        dc[k][2] = dc[k][2] * sgn

    mx = jnp.abs(dc[0][0])
    for k in range(KNN):
        for b in range(3):
            if k == 0 and b == 0:
                continue
            mx = jnp.maximum(mx, jnp.abs(dc[k][b]))

    inv = 1.0 / mx
    for b in range(3):
        out_ref[b] = dc[0][b] * inv * 0.5 + 0.5
    for j in range(3):
        out_ref[3 + j] = V[j][2]


def _spline_kernel(ps_ref, wf_ref, root_ref, bias_ref, out_ref):
    ps = ps_ref[...]
    n = ps.shape[0]
    p = jnp.clip(ps, 0.0, 1.0) * (KS - 1)
    i0f = jnp.clip(jnp.floor(p), 0.0, KS - 2.0)
    fr = p - i0f
    i0 = i0f.astype(jnp.int32)

    iota = jax.lax.broadcasted_iota(jnp.int32, (n, KS ** 3), 1)
    C = jnp.zeros((n, KS ** 3), dtype=jnp.float32)
    for b in range(8):
        basis = jnp.ones((n, 1), dtype=jnp.float32)
        wi = jnp.zeros((n, 1), dtype=jnp.int32)
        for d in range(3):
            bit = (b >> d) & 1
            frd = fr[:, d:d + 1]
            basis = basis * (frd if bit else (1.0 - frd))
            wi = wi + (i0[:, d:d + 1] + bit) * (KS ** d)
        C = C + basis * (iota == wi).astype(jnp.float32)

    msg = jax.lax.dot_general(
        C, wf_ref[...], (((1,), (0,)), ((), ())),
        preferred_element_type=jnp.float32)
    out_ref[...] = msg + root_ref[...] + bias_ref[...]


def _bn_kernel(o_ref, gamma_ref, beta_ref, out_ref):
    o = o_ref[...]
    mu = jnp.mean(o, axis=0, keepdims=True)
    dv = o - mu
    var = jnp.mean(dv * dv, axis=0, keepdims=True)
    out_ref[...] = dv * jax.lax.rsqrt(var + 1e-5) * gamma_ref[...] \
        + beta_ref[...]


def kernel(pos, edge_index, batch, W_spline, root, conv_bias, bn_gamma,
           bn_beta, W1, b1, W2, b2):
    pos3 = pos.reshape(B, P, 3)

    clusters = pl.pallas_call(
        _knn_clusters_kernel,
        out_shape=jax.ShapeDtypeStruct((B, P, 3 * KNN), jnp.float32),
        grid=(B,),
        in_specs=[pl.BlockSpec((1, P, 3), lambda b: (b, 0, 0))],
        out_specs=pl.BlockSpec((1, P, 3 * KNN), lambda b: (b, 0, 0)),
    )(pos3)

    cl_t = clusters.reshape(N, 3 * KNN).T.reshape(3 * KNN, 64, 128)

    frames = pl.pallas_call(
        _frames_kernel,
        out_shape=jax.ShapeDtypeStruct((6, 64, 128), jnp.float32),
    )(cl_t)

    ft = frames.reshape(6, N).T
    pseudo0 = ft[:, :3]
    v2 = ft[:, 3:]

    Wf = W_spline[:, 0, :]
    o = pl.pallas_call(
        _spline_kernel,
        out_shape=jax.ShapeDtypeStruct((N, F), jnp.float32),
        grid=(B,),
        in_specs=[
            pl.BlockSpec((P, 3), lambda b: (b, 0)),
            pl.BlockSpec((KS ** 3, F), lambda b: (0, 0)),
            pl.BlockSpec((1, F), lambda b: (0, 0)),
            pl.BlockSpec((1, F), lambda b: (0, 0)),
        ],
        out_specs=pl.BlockSpec((P, F), lambda b: (b, 0)),
    )(pseudo0, Wf, root.reshape(1, F), conv_bias.reshape(1, F))

    xn = pl.pallas_call(
        _bn_kernel,
        out_shape=jax.ShapeDtypeStruct((N, F), jnp.float32),
    )(o, bn_gamma.reshape(1, F), bn_beta.reshape(1, F))

    y = jax.nn.sigmoid(xn[:, :, None] * v2[:, None, :])
    ys = y.reshape(-1, P, F).mean(axis=1)
    y1 = jax.nn.elu(ys @ W1.T + b1)
    y2 = y1 @ W2.T + b2
    return jax.nn.log_softmax(y2, axis=1)


# speedup vs baseline: 28.1900x; 1.0253x over previous
import jax
import jax.numpy as jnp
from jax.experimental import pallas as pl
from jax.experimental.pallas import tpu as pltpu

B = 8
P = 1024
N = B * P
KNN = 20
L = 10
F = 64
KS = 5
NC = 40


def _knn_clusters_kernel(pos_ref, out_ref):
    x = pos_ref[0]
    row = jax.lax.broadcasted_iota(jnp.int32, (P, P), 0)
    col = jax.lax.broadcasted_iota(jnp.int32, (P, P), 1)
    d2 = jnp.zeros((P, P), dtype=jnp.float32)
    for d in range(3):
        xd = x[:, d:d + 1]
        diff = xd - xd.T
        d2 = d2 + diff * diff
    d2 = d2 + jnp.where(row == col, jnp.float32(1e10), jnp.float32(0.0))

    cols = col
    parts = []
    for _ in range(KNN):
        idx = jnp.argmin(d2, axis=1).astype(jnp.int32)
        onehot = (cols == idx[:, None]).astype(jnp.float32)
        nbr = jax.lax.dot_general(
            onehot, x, (((1,), (0,)), ((), ())),
            preferred_element_type=jnp.float32)
        parts.append(x - nbr)
        d2 = d2 + onehot * jnp.float32(1e10)
    out_ref[0] = jnp.concatenate(parts, axis=1)


def _frames_kernel(cl_ref, out_ref):
    c = [[cl_ref[k * 3 + d] for d in range(3)] for k in range(KNN)]

    def acc(a, b):
        s = c[0][a] * c[0][b]
        for k in range(1, L):
            s = s + c[k][a] * c[k][b]
        return s

    A = {}
    A[(0, 0)] = acc(0, 0); A[(1, 1)] = acc(1, 1); A[(2, 2)] = acc(2, 2)
    A[(0, 1)] = acc(0, 1); A[(0, 2)] = acc(0, 2); A[(1, 2)] = acc(1, 2)

    one = jnp.ones_like(A[(0, 0)])
    zero = jnp.zeros_like(one)
    V = [[one, zero, zero], [zero, one, zero], [zero, zero, one]]

    for _ in range(5):
        for (p, q) in ((0, 1), (0, 2), (1, 2)):
            r = 3 - p - q
            app = A[(p, p)]; aqq = A[(q, q)]; apq = A[(p, q)]
            apr = A[(min(p, r), max(p, r))]
            aqr = A[(min(q, r), max(q, r))]
            th = 0.5 * jnp.arctan2(2.0 * apq, aqq - app)
            cth = jnp.cos(th); sth = jnp.sin(th)
            cc = cth * cth; ss = sth * sth; cs = cth * sth
            new_pp = cc * app - 2.0 * cs * apq + ss * aqq
            new_qq = ss * app + 2.0 * cs * apq + cc * aqq
            new_pq = cs * (app - aqq) + (cc - ss) * apq
            new_pr = cth * apr - sth * aqr
            new_qr = sth * apr + cth * aqr
            A[(p, p)] = new_pp
            A[(q, q)] = new_qq
            A[(p, q)] = new_pq
            A[(min(p, r), max(p, r))] = new_pr
            A[(min(q, r), max(q, r))] = new_qr
            for i in range(3):
                vip = V[i][p]; viq = V[i][q]
                V[i][p] = cth * vip - sth * viq
                V[i][q] = sth * vip + cth * viq

    dc = [[c[k][0] * V[0][b] + c[k][1] * V[1][b] + c[k][2] * V[2][b]
           for b in range(3)] for k in range(KNN)]

    s2 = dc[0][2]
    for k in range(1, KNN):
        s2 = s2 + dc[k][2]
    sgn = jnp.sign(s2)
    for k in range(KNN):
        dc[k][2] = dc[k][2] * sgn

    mx = jnp.abs(dc[0][0])
    for k in range(KNN):
        for b in range(3):
            if k == 0 and b == 0:
                continue
            mx = jnp.maximum(mx, jnp.abs(dc[k][b]))

    inv = 1.0 / mx
    for b in range(3):
        out_ref[b] = dc[0][b] * inv * 0.5 + 0.5
    for j in range(3):
        out_ref[3 + j] = V[j][2]


def _spline_kernel(ps_ref, wf_ref, root_ref, bias_ref, out_ref):
    ps = ps_ref[...]
    n = ps.shape[0]
    p = jnp.clip(ps, 0.0, 1.0) * (KS - 1)
    i0f = jnp.clip(jnp.floor(p), 0.0, KS - 2.0)
    fr = p - i0f
    i0 = i0f.astype(jnp.int32)

    iota = jax.lax.broadcasted_iota(jnp.int32, (n, KS ** 3), 1)
    C = jnp.zeros((n, KS ** 3), dtype=jnp.float32)
    for b in range(8):
        basis = jnp.ones((n, 1), dtype=jnp.float32)
        wi = jnp.zeros((n, 1), dtype=jnp.int32)
        for d in range(3):
            bit = (b >> d) & 1
            frd = fr[:, d:d + 1]
            basis = basis * (frd if bit else (1.0 - frd))
            wi = wi + (i0[:, d:d + 1] + bit) * (KS ** d)
        C = C + basis * (iota == wi).astype(jnp.float32)

    msg = jax.lax.dot_general(
        C, wf_ref[...], (((1,), (0,)), ((), ())),
        preferred_element_type=jnp.float32)
    out_ref[...] = msg + root_ref[...] + bias_ref[...]


def _bn_kernel(o_ref, gamma_ref, beta_ref, out_ref):
    o = o_ref[...]
    mu = jnp.mean(o, axis=0, keepdims=True)
    dv = o - mu
    var = jnp.mean(dv * dv, axis=0, keepdims=True)
    out_ref[...] = dv * jax.lax.rsqrt(var + 1e-5) * gamma_ref[...] \
        + beta_ref[...]


def kernel(pos, edge_index, batch, W_spline, root, conv_bias, bn_gamma,
           bn_beta, W1, b1, W2, b2):
    pos3 = pos.reshape(B, P, 3)

    clusters = pl.pallas_call(
        _knn_clusters_kernel,
        out_shape=jax.ShapeDtypeStruct((B, P, 3 * KNN), jnp.float32),
        grid=(B,),
        in_specs=[pl.BlockSpec((1, P, 3), lambda b: (b, 0, 0))],
        out_specs=pl.BlockSpec((1, P, 3 * KNN), lambda b: (b, 0, 0)),
        compiler_params=pltpu.CompilerParams(
            dimension_semantics=("parallel",)),
    )(pos3)

    cl_t = clusters.reshape(N, 3 * KNN).T.reshape(3 * KNN, 64, 128)

    frames = pl.pallas_call(
        _frames_kernel,
        out_shape=jax.ShapeDtypeStruct((6, 64, 128), jnp.float32),
    )(cl_t)

    ft = frames.reshape(6, N).T
    pseudo0 = ft[:, :3]
    v2 = ft[:, 3:]

    Wf = W_spline[:, 0, :]
    o = pl.pallas_call(
        _spline_kernel,
        out_shape=jax.ShapeDtypeStruct((N, F), jnp.float32),
        grid=(B,),
        in_specs=[
            pl.BlockSpec((P, 3), lambda b: (b, 0)),
            pl.BlockSpec((KS ** 3, F), lambda b: (0, 0)),
            pl.BlockSpec((1, F), lambda b: (0, 0)),
            pl.BlockSpec((1, F), lambda b: (0, 0)),
        ],
        out_specs=pl.BlockSpec((P, F), lambda b: (b, 0)),
    )(pseudo0, Wf, root.reshape(1, F), conv_bias.reshape(1, F))

    xn = pl.pallas_call(
        _bn_kernel,
        out_shape=jax.ShapeDtypeStruct((N, F), jnp.float32),
    )(o, bn_gamma.reshape(1, F), bn_beta.reshape(1, F))

    y = jax.nn.sigmoid(xn[:, :, None] * v2[:, None, :])
    ys = y.reshape(-1, P, F).mean(axis=1)
    y1 = jax.nn.elu(ys @ W1.T + b1)
    y2 = y1 @ W2.T + b2
    return jax.nn.log_softmax(y2, axis=1)
